# baseline (speedup 1.0000x reference)
import functools

import jax
import jax.numpy as jnp
from jax import lax
from jax.experimental import pallas as pl
from jax.experimental.pallas import tpu as pltpu
from jax.experimental.pallas import tpu_sc as plsc

L = 16


def _make_sc_kernel(B, C, J, rows_per_w, rchunk):
  nchunks = rows_per_w // rchunk
  mesh = plsc.VectorSubcoreMesh(core_axis_name="c", subcore_axis_name="s")

  @functools.partial(
      pl.kernel,
      mesh=mesh,
      out_type=jax.ShapeDtypeStruct((B * J,), jnp.float32),
      scratch_types=[
          pltpu.VMEM((J,), jnp.int32),
          pltpu.VMEM((rchunk * C,), jnp.float32),
          pltpu.VMEM((rchunk * J,), jnp.float32),
      ],
      compiler_params=pltpu.CompilerParams(needs_layout_passes=False),
  )
  def k(x_hbm, idx_hbm, out_hbm, idx_v, xin_v, oout_v):
    wid = lax.axis_index("s") * 2 + lax.axis_index("c")
    base = wid * rows_per_w
    pltpu.sync_copy(idx_hbm, idx_v)

    def chunk_body(ci, _):
      r0 = base + ci * rchunk
      pltpu.sync_copy(x_hbm.at[pl.ds(r0 * C, rchunk * C)], xin_v)
      for j in range(J // L):
        iv = idx_v[pl.ds(j * L, L)]

        @plsc.parallel_loop(0, rchunk, 1, unroll=4)
        def row_body(r):
          flat = jnp.broadcast_to(r * C, (L,)).astype(jnp.int32) + iv
          vals = plsc.load_gather(xin_v, [flat])
          oout_v[pl.ds(r * J + j * L, L)] = vals

      pltpu.sync_copy(oout_v, out_hbm.at[pl.ds(r0 * J, rchunk * J)])

    lax.fori_loop(0, nchunks, chunk_body, None)

  return k


def kernel(x, hierarchy_mapping_idx):
  B, C = x.shape
  J = hierarchy_mapping_idx.shape[0]
  rows_per_w = B // 32
  rchunk = 32
  k = _make_sc_kernel(B, C, J, rows_per_w, rchunk)
  out_flat = k(x.reshape(B * C), hierarchy_mapping_idx.astype(jnp.int32))
  return out_flat.reshape(B, J)
